# baseline (speedup 1.0000x reference)
import functools

import jax
import jax.numpy as jnp
from jax import lax
from jax.experimental import pallas as pl
from jax.experimental.pallas import tpu as pltpu
from jax.experimental.pallas import tpu_sc as plsc

B = 16384
NF = 26
ND = 13
VOCAB = 1000000
NC = 2
NSUB = 16
L = 16
NW = NC * NSUB
NB = B // NW
CH = 128
NCH = NB // CH

_mesh = plsc.VectorSubcoreMesh(core_axis_name="c", subcore_axis_name="s")

RW = 4992
VOCAB_AL = (VOCAB // 128) * 128
RNCH = VOCAB_AL // RW
RTOFF = RNCH * RW
RT = VOCAB_AL - RTOFF
RREST = VOCAB - VOCAB_AL


@functools.partial(
    pl.kernel,
    out_type=jax.ShapeDtypeStruct((NF * VOCAB,), jnp.float32),
    mesh=_mesh,
    scratch_types=[pltpu.VMEM((8, RW), jnp.float32),
                   pltpu.VMEM((8 * RW,), jnp.float32),
                   pltpu.SemaphoreType.DMA],
)
def _relayout_sc(tbl_hbm, tail_hbm, out_hbm, buf_t, buf_u, sem):
    wid = lax.axis_index("s") * NC + lax.axis_index("c")

    def chunk(r0, nr, c0, w):
        pltpu.sync_copy(tbl_hbm.at[pl.ds(r0, nr), pl.ds(c0, w)],
                        buf_t.at[pl.ds(0, nr), pl.ds(0, w)])
        for rp in range(nr // 2):
            a, b = 2 * rp, 2 * rp + 1

            def cp(i, carry, a=a, b=b):
                for u in range(12):
                    off = i * 192 + u * 16
                    buf_u[pl.ds(a * RW + off, L)] = buf_t[a, pl.ds(off, L)]
                    buf_u[pl.ds(b * RW + off, L)] = buf_t[b, pl.ds(off, L)]
                return carry

            lax.fori_loop(0, w // 192, cp, 0)
        for r in range(nr):
            pltpu.async_copy(
                buf_u.at[pl.ds(r * RW, w)],
                out_hbm.at[pl.ds((r0 + r) * VOCAB + c0, w)],
                sem)
        for r in range(nr):
            pltpu.make_async_copy(
                out_hbm.at[pl.ds(r * VOCAB, w)],
                buf_u.at[pl.ds(r * RW, w)],
                sem).wait()

    for rb in range(4):
        nr = 8 if rb < 3 else 2
        r0 = rb * 8

        def body(k, carry, nr=nr, r0=r0):
            ch = wid + k * NW
            c0 = pl.multiple_of(ch * RW, 128)
            chunk(r0, nr, c0, RW)
            return carry

        lax.fori_loop(0, 6 + (wid < RNCH - 6 * NW).astype(jnp.int32), body, 0)

        @pl.when(wid == rb)
        def _(nr=nr, r0=r0):
            chunk(r0, nr, RTOFF, RT)

    @pl.when(wid >= NW - NF)
    def _():
        f = wid - (NW - NF)
        pltpu.sync_copy(tail_hbm.at[pl.ds(f * RREST, RREST)],
                        buf_u.at[pl.ds(0, RREST)])
        pltpu.sync_copy(buf_u.at[pl.ds(0, RREST)],
                        out_hbm.at[pl.ds(f * VOCAB + VOCAB_AL, RREST)])


@functools.partial(
    pl.kernel,
    out_type=jax.ShapeDtypeStruct((B,), jnp.float32),
    mesh=_mesh,
    scratch_types=[
        pltpu.VMEM((NF, NB), jnp.float32),
        pltpu.VMEM((NF, NB), jnp.int32),
        pltpu.VMEM((NF, NB), jnp.float32),
        pltpu.VMEM((ND, NB), jnp.float32),
        pltpu.VMEM((ND, L), jnp.float32),
        pltpu.VMEM((NB,), jnp.float32),
        pltpu.SemaphoreType.DMA,
    ],
)
def _linear_sc(xi_hbm, xd_hbm, tbl_hbm, w_hbm, out_hbm,
               xi_v, idx_v, vals_v, xd_v, w_v, acc_v, sem):
    wid = lax.axis_index("s") * NC + lax.axis_index("c")
    base = wid * NB

    pltpu.sync_copy(xi_hbm.at[:, pl.ds(base, NB)], xi_v)
    pltpu.sync_copy(xd_hbm.at[:, pl.ds(base, NB)], xd_v)
    pltpu.sync_copy(w_hbm, w_v)

    def idx_body(c, carry):
        off = c * L
        for f in range(NF):
            v = xi_v[f, pl.ds(off, L)]
            idx_v[f, pl.ds(off, L)] = v.astype(jnp.int32) + f * VOCAB
        return carry

    lax.fori_loop(0, NB // L, idx_body, 0)

    copies = []
    for f in range(NF):
        for k in range(NCH):
            copies.append(pltpu.async_copy(
                tbl_hbm.at[idx_v.at[f, pl.ds(k * CH, CH)]],
                vals_v.at[f, pl.ds(k * CH, CH)],
                sem))
    for cp in copies:
        cp.wait()

    def red_body(c, carry):
        off = c * L
        acc = vals_v[0, pl.ds(off, L)]
        for f in range(1, NF):
            acc = acc + vals_v[f, pl.ds(off, L)]
        for d in range(ND):
            acc = acc + xd_v[d, pl.ds(off, L)] * w_v[d]
        acc_v[pl.ds(off, L)] = acc
        return carry

    lax.fori_loop(0, NB // L, red_body, 0)

    pltpu.sync_copy(acc_v, out_hbm.at[pl.ds(base, NB)])


def kernel(X, tables, W):
    xi_t = X[:, :NF].T
    xd_t = X[:, NF:].T
    tail = tables[:, VOCAB_AL:].reshape(-1)
    tbl = _relayout_sc(tables, tail)
    w_rep = jnp.broadcast_to(W.reshape(ND, 1), (ND, L))
    out = _linear_sc(xi_t, xd_t, tbl, w_rep)
    return out.reshape(B, 1)


# speedup vs baseline: 1.0898x; 1.0898x over previous
import functools

import jax
import jax.numpy as jnp
from jax import lax
from jax.experimental import pallas as pl
from jax.experimental.pallas import tpu as pltpu
from jax.experimental.pallas import tpu_sc as plsc

B = 16384
NF = 26
ND = 13
VOCAB = 1000000
NC = 2
NSUB = 16
L = 16
NW = NC * NSUB
NB = B // NW
CH = 128
NCH = NB // CH

_mesh = plsc.VectorSubcoreMesh(core_axis_name="c", subcore_axis_name="s")

RW = 4992
VOCAB_AL = (VOCAB // 128) * 128
RNCH = VOCAB_AL // RW
RTOFF = RNCH * RW
RT = VOCAB_AL - RTOFF
RREST = VOCAB - VOCAB_AL


@functools.partial(
    pl.kernel,
    out_type=jax.ShapeDtypeStruct((NF * VOCAB,), jnp.float32),
    mesh=_mesh,
    scratch_types=[pltpu.VMEM((8, RW), jnp.float32),
                   pltpu.VMEM((8 * RW,), jnp.float32),
                   pltpu.SemaphoreType.DMA],
)
def _relayout_sc(tbl_hbm, tail_hbm, out_hbm, buf_t, buf_u, sem):
    wid = lax.axis_index("s") * NC + lax.axis_index("c")

    def chunk(r0, nr, c0, w):
        pltpu.sync_copy(tbl_hbm.at[pl.ds(r0, nr), pl.ds(c0, w)],
                        buf_t.at[pl.ds(0, nr), pl.ds(0, w)])
        for r in range(nr):
            def cp(i, carry, r=r):
                for u in range(24):
                    off = i * 384 + u * 16
                    buf_u[pl.ds(r * RW + off, L)] = buf_t[r, pl.ds(off, L)]
                return carry

            lax.fori_loop(0, w // 384, cp, 0)
            pltpu.async_copy(
                buf_u.at[pl.ds(r * RW, w)],
                out_hbm.at[pl.ds((r0 + r) * VOCAB + c0, w)],
                sem)
        for r in range(nr):
            pltpu.make_async_copy(
                out_hbm.at[pl.ds(r * VOCAB, w)],
                buf_u.at[pl.ds(r * RW, w)],
                sem).wait()

    for rb in range(4):
        nr = 8 if rb < 3 else 2
        r0 = rb * 8

        def body(k, carry, nr=nr, r0=r0):
            ch = wid + k * NW
            c0 = pl.multiple_of(ch * RW, 128)
            chunk(r0, nr, c0, RW)
            return carry

        lax.fori_loop(0, 6 + (wid < RNCH - 6 * NW).astype(jnp.int32), body, 0)

        @pl.when(wid == rb)
        def _(nr=nr, r0=r0):
            chunk(r0, nr, RTOFF, RT)

    @pl.when(wid >= NW - NF)
    def _():
        f = wid - (NW - NF)
        pltpu.sync_copy(tail_hbm.at[pl.ds(f * RREST, RREST)],
                        buf_u.at[pl.ds(0, RREST)])
        pltpu.sync_copy(buf_u.at[pl.ds(0, RREST)],
                        out_hbm.at[pl.ds(f * VOCAB + VOCAB_AL, RREST)])


@functools.partial(
    pl.kernel,
    out_type=jax.ShapeDtypeStruct((B,), jnp.float32),
    mesh=_mesh,
    scratch_types=[
        pltpu.VMEM((NF, NB), jnp.float32),
        pltpu.VMEM((NF, NB), jnp.int32),
        pltpu.VMEM((NF, NB), jnp.float32),
        pltpu.VMEM((ND, NB), jnp.float32),
        pltpu.VMEM((ND, L), jnp.float32),
        pltpu.VMEM((NB,), jnp.float32),
        pltpu.SemaphoreType.DMA,
    ],
)
def _linear_sc(xi_hbm, xd_hbm, tbl_hbm, w_hbm, out_hbm,
               xi_v, idx_v, vals_v, xd_v, w_v, acc_v, sem):
    wid = lax.axis_index("s") * NC + lax.axis_index("c")
    base = wid * NB

    pltpu.sync_copy(xi_hbm.at[:, pl.ds(base, NB)], xi_v)
    pltpu.sync_copy(xd_hbm.at[:, pl.ds(base, NB)], xd_v)
    pltpu.sync_copy(w_hbm, w_v)

    def idx_body(c, carry):
        off = c * L
        for f in range(NF):
            v = xi_v[f, pl.ds(off, L)]
            idx_v[f, pl.ds(off, L)] = v.astype(jnp.int32) + f * VOCAB
        return carry

    lax.fori_loop(0, NB // L, idx_body, 0)

    copies = []
    for f in range(NF):
        for k in range(NCH):
            copies.append(pltpu.async_copy(
                tbl_hbm.at[idx_v.at[f, pl.ds(k * CH, CH)]],
                vals_v.at[f, pl.ds(k * CH, CH)],
                sem))
    for cp in copies:
        cp.wait()

    def red_body(c, carry):
        off = c * L
        acc = vals_v[0, pl.ds(off, L)]
        for f in range(1, NF):
            acc = acc + vals_v[f, pl.ds(off, L)]
        for d in range(ND):
            acc = acc + xd_v[d, pl.ds(off, L)] * w_v[d]
        acc_v[pl.ds(off, L)] = acc
        return carry

    lax.fori_loop(0, NB // L, red_body, 0)

    pltpu.sync_copy(acc_v, out_hbm.at[pl.ds(base, NB)])


def kernel(X, tables, W):
    xi_t = X[:, :NF].T
    xd_t = X[:, NF:].T
    tail = tables[:, VOCAB_AL:].reshape(-1)
    tbl = _relayout_sc(tables, tail)
    w_rep = jnp.broadcast_to(W.reshape(ND, 1), (ND, L))
    out = _linear_sc(xi_t, xd_t, tbl, w_rep)
    return out.reshape(B, 1)


# speedup vs baseline: 2.2830x; 2.0950x over previous
import functools

import jax
import jax.numpy as jnp
from jax import lax
from jax.experimental import pallas as pl
from jax.experimental.pallas import tpu as pltpu
from jax.experimental.pallas import tpu_sc as plsc

B = 16384
NF = 26
ND = 13
VOCAB = 1000000
NC = 2
NSUB = 16
L = 16
NW = NC * NSUB
NB = B // NW
CH = 128
NCH = NB // CH

_mesh = plsc.VectorSubcoreMesh(core_axis_name="c", subcore_axis_name="s")

RW = 4992
VOCAB_AL = (VOCAB // 128) * 128
RNCH = VOCAB_AL // RW
RTOFF = RNCH * RW
RT = VOCAB_AL - RTOFF
RREST = VOCAB - VOCAB_AL


@functools.partial(
    pl.kernel,
    out_type=jax.ShapeDtypeStruct((NF * VOCAB,), jnp.float32),
    mesh=_mesh,
    scratch_types=[pltpu.VMEM((8, RW), jnp.float32),
                   pltpu.VMEM((8 * RW,), jnp.float32),
                   pltpu.SemaphoreType.DMA],
)
def _relayout_sc(tbl_hbm, tail_hbm, out_hbm, buf_t, buf_u, sem):
    wid = lax.axis_index("s") * NC + lax.axis_index("c")

    def chunk(r0, nr, c0, w):
        pltpu.sync_copy(tbl_hbm.at[pl.ds(r0, nr), pl.ds(c0, w)],
                        buf_t.at[pl.ds(0, nr), pl.ds(0, w)])
        for r in range(nr):
            @plsc.parallel_loop(0, w, L, unroll=8)
            def _(i, r=r):
                buf_u[pl.ds(r * RW + i, L)] = buf_t[r, pl.ds(i, L)]

            pltpu.async_copy(
                buf_u.at[pl.ds(r * RW, w)],
                out_hbm.at[pl.ds((r0 + r) * VOCAB + c0, w)],
                sem)
        for r in range(nr):
            pltpu.make_async_copy(
                out_hbm.at[pl.ds(r * VOCAB, w)],
                buf_u.at[pl.ds(r * RW, w)],
                sem).wait()

    for rb in range(4):
        nr = 8 if rb < 3 else 2
        r0 = rb * 8

        def body(k, carry, nr=nr, r0=r0):
            ch = wid + k * NW
            c0 = pl.multiple_of(ch * RW, 128)
            chunk(r0, nr, c0, RW)
            return carry

        lax.fori_loop(0, 6 + (wid < RNCH - 6 * NW).astype(jnp.int32), body, 0)

        @pl.when(wid == rb)
        def _(nr=nr, r0=r0):
            chunk(r0, nr, RTOFF, RT)

    @pl.when(wid >= NW - NF)
    def _():
        f = wid - (NW - NF)
        pltpu.sync_copy(tail_hbm.at[pl.ds(f * RREST, RREST)],
                        buf_u.at[pl.ds(0, RREST)])
        pltpu.sync_copy(buf_u.at[pl.ds(0, RREST)],
                        out_hbm.at[pl.ds(f * VOCAB + VOCAB_AL, RREST)])


@functools.partial(
    pl.kernel,
    out_type=jax.ShapeDtypeStruct((B,), jnp.float32),
    mesh=_mesh,
    scratch_types=[
        pltpu.VMEM((NF, NB), jnp.float32),
        pltpu.VMEM((NF, NB), jnp.int32),
        pltpu.VMEM((NF, NB), jnp.float32),
        pltpu.VMEM((ND, NB), jnp.float32),
        pltpu.VMEM((ND, L), jnp.float32),
        pltpu.VMEM((NB,), jnp.float32),
        pltpu.SemaphoreType.DMA,
    ],
)
def _linear_sc(xi_hbm, xd_hbm, tbl_hbm, w_hbm, out_hbm,
               xi_v, idx_v, vals_v, xd_v, w_v, acc_v, sem):
    wid = lax.axis_index("s") * NC + lax.axis_index("c")
    base = wid * NB

    pltpu.sync_copy(xi_hbm.at[:, pl.ds(base, NB)], xi_v)
    pltpu.sync_copy(xd_hbm.at[:, pl.ds(base, NB)], xd_v)
    pltpu.sync_copy(w_hbm, w_v)

    def idx_body(c, carry):
        off = c * L
        for f in range(NF):
            v = xi_v[f, pl.ds(off, L)]
            idx_v[f, pl.ds(off, L)] = v.astype(jnp.int32) + f * VOCAB
        return carry

    lax.fori_loop(0, NB // L, idx_body, 0)

    copies = []
    for f in range(NF):
        for k in range(NCH):
            copies.append(pltpu.async_copy(
                tbl_hbm.at[idx_v.at[f, pl.ds(k * CH, CH)]],
                vals_v.at[f, pl.ds(k * CH, CH)],
                sem))
    for cp in copies:
        cp.wait()

    def red_body(c, carry):
        off = c * L
        acc = vals_v[0, pl.ds(off, L)]
        for f in range(1, NF):
            acc = acc + vals_v[f, pl.ds(off, L)]
        for d in range(ND):
            acc = acc + xd_v[d, pl.ds(off, L)] * w_v[d]
        acc_v[pl.ds(off, L)] = acc
        return carry

    lax.fori_loop(0, NB // L, red_body, 0)

    pltpu.sync_copy(acc_v, out_hbm.at[pl.ds(base, NB)])


def kernel(X, tables, W):
    xi_t = X[:, :NF].T
    xd_t = X[:, NF:].T
    tail = tables[:, VOCAB_AL:].reshape(-1)
    tbl = _relayout_sc(tables, tail)
    w_rep = jnp.broadcast_to(W.reshape(ND, 1), (ND, L))
    out = _linear_sc(xi_t, xd_t, tbl, w_rep)
    return out.reshape(B, 1)
